# baseline (speedup 1.0000x reference)
import functools

import jax
import jax.numpy as jnp
from jax import lax
from jax.experimental import pallas as pl
from jax.experimental.pallas import tpu as pltpu
from jax.experimental.pallas import tpu_sc as plsc

VOCAB = 100000
EMBED = 64
B = 1024
L = 200
SCALE = 8.0

NC = 2
NS = 16
NW = NC * NS
EPW = B // NW
LPC = 4
NCH = L // LPC
NBUF = 3

_mesh = plsc.VectorSubcoreMesh(core_axis_name="c", subcore_axis_name="s")


@functools.partial(
    pl.kernel,
    mesh=_mesh,
    out_type=jax.ShapeDtypeStruct((L, 8, 8, 8, 128), jnp.float32),
    scratch_types=[
        pltpu.VMEM((EPW, L), jnp.int32),
        pltpu.VMEM((NCH, 4 * EPW), jnp.int32),
        [pltpu.VMEM((4 * EPW, EMBED), jnp.float32)] * NBUF,
        [pltpu.VMEM((LPC, 8, 8, EPW), jnp.float32)] * NBUF,
        [pltpu.SemaphoreType.DMA] * NBUF,
        [pltpu.SemaphoreType.DMA] * NBUF,
    ],
    compiler_params=pltpu.CompilerParams(
        use_tc_tiling_on_sc=False, needs_layout_passes=False),
)
def _embed_gather(idx_hbm, table_hbm, out_hbm, idx_v, idxT, gbufs, obufs,
                  gsems, osems):
    wid = lax.axis_index("s") * NC + lax.axis_index("c")
    bc = wid // 4
    bq = wid % 4
    pltpu.sync_copy(idx_hbm.at[pl.ds(wid * EPW, EPW)], idx_v)

    iota = jax.lax.iota(jnp.int32, 16)

    def build_body(l, _):
        r = lax.div(l, LPC)
        c = lax.mul(lax.rem(l, LPC), EPW)
        for h in range(2):
            rows = iota + (16 * h)
            cols = jnp.full((16,), l, jnp.int32)
            v = plsc.load_gather(idx_v, [rows, cols])
            idxT[r, pl.ds(c + 16 * h, 16)] = v
        return 0

    lax.fori_loop(0, L, build_body, 0)

    def repack(gbuf, obuf):
        def lq_body(lq, _):
            base = lax.mul(lq, EPW)
            for h in range(2):
                rows = iota + 16 * h

                @plsc.parallel_loop(0, EMBED, step=1, unroll=4)
                def _(e):
                    cols = jnp.full((16,), e, jnp.int32)
                    v = plsc.load_gather(gbuf, [base + rows, cols]) * SCALE
                    obuf[lq, lax.shift_right_logical(e, 3), lax.rem(e, 8),
                         pl.ds(16 * h, 16)] = v
            return 0

        lax.fori_loop(0, LPC, lq_body, 0)

    ghandles = {}
    ohandles = {}

    for g in range(NCH + 1):
        b = g % NBUF
        if g < NCH:
            if g >= NBUF:
                ohandles[g - NBUF].wait()
            ghandles[g] = pltpu.async_copy(
                table_hbm.at[idxT.at[g]], gbufs[b], gsems[b])
        if g >= 1:
            gp = g - 1
            bp = gp % NBUF
            ghandles[gp].wait()
            repack(gbufs[bp], obufs[bp])
            ohandles[gp] = pltpu.async_copy(
                obufs[bp],
                out_hbm.at[pl.ds(gp * LPC, LPC), :, bc, :,
                           pl.ds(bq * EPW, EPW)],
                osems[bp])

    for g in range(NCH - NBUF, NCH):
        ohandles[g].wait()


def kernel(tokens, input_embedding_table):
    out = _embed_gather(tokens.astype(jnp.int32), input_embedding_table)
    return (out.transpose(2, 4, 0, 1, 3)
               .reshape(B, L, EMBED))


# speedup vs baseline: 1.0149x; 1.0149x over previous
import functools

import jax
import jax.numpy as jnp
from jax import lax
from jax.experimental import pallas as pl
from jax.experimental.pallas import tpu as pltpu
from jax.experimental.pallas import tpu_sc as plsc

VOCAB = 100000
EMBED = 64
B = 1024
L = 200
SCALE = 8.0

NC = 2
NS = 16
NW = NC * NS
EPW = B // NW
LPC = 4
NCH = L // LPC
NBUF = 3

_mesh = plsc.VectorSubcoreMesh(core_axis_name="c", subcore_axis_name="s")


@functools.partial(
    pl.kernel,
    mesh=_mesh,
    out_type=jax.ShapeDtypeStruct((L, 8, 8, 8, 128), jnp.float32),
    scratch_types=[
        pltpu.VMEM((EPW, L), jnp.int32),
        pltpu.VMEM((NCH, 4 * EPW), jnp.int32),
        [pltpu.VMEM((4 * EPW, EMBED), jnp.float32)] * NBUF,
        [pltpu.VMEM((LPC, 8, 8, EPW), jnp.float32)] * NBUF,
        [pltpu.SemaphoreType.DMA] * NBUF,
        [pltpu.SemaphoreType.DMA] * NBUF,
    ],
    compiler_params=pltpu.CompilerParams(
        use_tc_tiling_on_sc=False, needs_layout_passes=False),
)
def _embed_gather(idx_hbm, table_hbm, out_hbm, idx_v, idxT, gbufs, obufs,
                  gsems, osems):
    wid = lax.axis_index("s") * NC + lax.axis_index("c")
    bc = wid // 4
    bq = wid % 4
    pltpu.sync_copy(idx_hbm.at[pl.ds(wid * EPW, EPW)], idx_v)

    iota = jax.lax.iota(jnp.int32, 16)

    def build_body(l, _):
        r = lax.div(l, LPC)
        c = lax.mul(lax.rem(l, LPC), EPW)
        for h in range(2):
            rows = iota + (16 * h)
            cols = jnp.full((16,), l, jnp.int32)
            v = plsc.load_gather(idx_v, [rows, cols])
            idxT[r, pl.ds(c + 16 * h, 16)] = v
        return 0

    lax.fori_loop(0, L, build_body, 0)

    def repack(gbuf, obuf):
        def lq_body(lq, _):
            base = lax.mul(lq, EPW)
            for h in range(2):
                rows = iota + (base + 16 * h)

                @plsc.parallel_loop(0, EMBED, step=1, unroll=4,
                                    carry=jnp.zeros((16,), jnp.int32))
                def _(e, cols):
                    v = plsc.load_gather(gbuf, [rows, cols]) * SCALE
                    obuf[lq, lax.shift_right_logical(e, 3), lax.rem(e, 8),
                         pl.ds(16 * h, 16)] = v
                    return cols + 1
            return 0

        lax.fori_loop(0, LPC, lq_body, 0)

    ghandles = {}
    ohandles = {}

    for g in range(NCH + 1):
        b = g % NBUF
        if g < NCH:
            if g >= NBUF:
                ohandles[g - NBUF].wait()
            ghandles[g] = pltpu.async_copy(
                table_hbm.at[idxT.at[g]], gbufs[b], gsems[b])
        if g >= 1:
            gp = g - 1
            bp = gp % NBUF
            ghandles[gp].wait()
            repack(gbufs[bp], obufs[bp])
            ohandles[gp] = pltpu.async_copy(
                obufs[bp],
                out_hbm.at[pl.ds(gp * LPC, LPC), :, bc, :,
                           pl.ds(bq * EPW, EPW)],
                osems[bp])

    for g in range(NCH - NBUF, NCH):
        ohandles[g].wait()


def kernel(tokens, input_embedding_table):
    out = _embed_gather(tokens.astype(jnp.int32), input_embedding_table)
    return (out.transpose(2, 4, 0, 1, 3)
               .reshape(B, L, EMBED))


# speedup vs baseline: 1.8700x; 1.8426x over previous
import functools

import jax
import jax.numpy as jnp
from jax import lax
from jax.experimental import pallas as pl
from jax.experimental.pallas import tpu as pltpu
from jax.experimental.pallas import tpu_sc as plsc

VOCAB = 100000
EMBED = 64
B = 1024
L = 200
SCALE = 8.0

NC = 2
NS = 16
NW = NC * NS
EPW = B // NW
LPC = 4
NCH = L // LPC
NBUF = 3

_mesh = plsc.VectorSubcoreMesh(core_axis_name="c", subcore_axis_name="s")


@functools.partial(
    pl.kernel,
    mesh=_mesh,
    out_type=jax.ShapeDtypeStruct((L, 8, 8, 8, 128), jnp.float32),
    scratch_types=[
        pltpu.VMEM((EPW, L), jnp.int32),
        pltpu.VMEM((NCH, 4 * EPW), jnp.int32),
        [pltpu.VMEM((4 * EPW, EMBED), jnp.float32)] * NBUF,
        [pltpu.VMEM((LPC, 8, 8, EPW + 1), jnp.float32)] * NBUF,
        [pltpu.SemaphoreType.DMA] * NBUF,
        [pltpu.SemaphoreType.DMA] * NBUF,
    ],
    compiler_params=pltpu.CompilerParams(
        use_tc_tiling_on_sc=False, needs_layout_passes=False),
)
def _embed_gather(idx_hbm, table_hbm, out_hbm, idx_v, idxT, gbufs, obufs,
                  gsems, osems):
    wid = lax.axis_index("s") * NC + lax.axis_index("c")
    bc = wid // 4
    bq = wid % 4
    pltpu.sync_copy(idx_hbm.at[pl.ds(wid * EPW, EPW)], idx_v)

    iota = jax.lax.iota(jnp.int32, 16)

    def build_body(l, _):
        r = lax.div(l, LPC)
        c = lax.mul(lax.rem(l, LPC), EPW)
        for h in range(2):
            rows = iota + (16 * h)
            cols = jnp.full((16,), l, jnp.int32)
            v = plsc.load_gather(idx_v, [rows, cols])
            idxT[r, pl.ds(c + 16 * h, 16)] = v
        return 0

    lax.fori_loop(0, L, build_body, 0)

    er_v = [lax.shift_right_logical(iota + 16 * j, 3)
            for j in range(EMBED // 16)]
    ei_v = [lax.rem(iota + 16 * j, 8) for j in range(EMBED // 16)]

    def repack(gbuf, obuf):
        @plsc.parallel_loop(0, LPC * EPW, step=1, unroll=2)
        def _(t):
            lqv = jnp.full((16,), lax.shift_right_logical(t, 5), jnp.int32)
            blv = jnp.full((16,), lax.rem(t, EPW), jnp.int32)
            for j in range(EMBED // 16):
                v = gbuf[t, pl.ds(16 * j, 16)] * SCALE
                plsc.store_scatter(obuf, [lqv, er_v[j], ei_v[j], blv], v)

    ghandles = {}
    ohandles = {}

    for g in range(NCH + 1):
        b = g % NBUF
        if g < NCH:
            if g >= NBUF:
                ohandles[g - NBUF].wait()
            ghandles[g] = pltpu.async_copy(
                table_hbm.at[idxT.at[g]], gbufs[b], gsems[b])
        if g >= 1:
            gp = g - 1
            bp = gp % NBUF
            ghandles[gp].wait()
            repack(gbufs[bp], obufs[bp])
            ohandles[gp] = pltpu.async_copy(
                obufs[bp].at[:, :, :, pl.ds(0, EPW)],
                out_hbm.at[pl.ds(gp * LPC, LPC), :, bc, :,
                           pl.ds(bq * EPW, EPW)],
                osems[bp])

    for g in range(NCH - NBUF, NCH):
        ohandles[g].wait()


def kernel(tokens, input_embedding_table):
    out = _embed_gather(tokens.astype(jnp.int32), input_embedding_table)
    return (out.transpose(2, 4, 0, 1, 3)
               .reshape(B, L, EMBED))
